# baseline (speedup 1.0000x reference)
import functools

import jax
import jax.numpy as jnp
from jax.experimental import pallas as pl
from jax.experimental.pallas import tpu as pltpu

_HIGH = jax.lax.Precision.HIGHEST


def _tc_body(rand_ref, patches_ref, w_ref, b_ref, pos_ref, mt_ref,
             ue_ref, me_ref, up_ref, mi_ref, ui_ref, *, num_mask):
    tb, p = rand_ref.shape
    a = patches_ref.shape[2]
    d = w_ref.shape[1]
    num_unmask = p - num_mask

    rand = rand_ref[...]

    rj = rand[:, :, None]
    rk = rand[:, None, :]
    ij = jax.lax.broadcasted_iota(jnp.int32, (tb, p, p), 1)
    ik = jax.lax.broadcasted_iota(jnp.int32, (tb, p, p), 2)
    before = (rk < rj) | ((rk == rj) & (ik < ij))
    ranks = jnp.sum(before.astype(jnp.int32), axis=2)

    eqr = ranks[:, None, :] == jax.lax.broadcasted_iota(
        jnp.int32, (tb, p, p), 1)
    inv = jnp.sum(eqr.astype(jnp.int32) *
                  jax.lax.broadcasted_iota(jnp.int32, (tb, p, p), 2), axis=2)

    mask_idx = inv[:, :num_mask]
    unmask_idx = inv[:, num_mask:]
    mi_ref[...] = mask_idx
    ui_ref[...] = unmask_idx

    patches = patches_ref[...]
    proj = jnp.dot(patches.reshape(tb * p, a), w_ref[...],
                   preferred_element_type=jnp.float32, precision=_HIGH)
    pe = proj.reshape(tb, p, d) + b_ref[...][None] + pos_ref[...][None]

    iota_p = jax.lax.broadcasted_iota(jnp.int32, (tb, p), 1)
    for i in range(num_unmask):
        sel = (unmask_idx[:, i][:, None] == iota_p).astype(jnp.float32)
        ue_ref[:, i, :] = jnp.sum(sel[:, :, None] * pe, axis=1)

    iota3 = jax.lax.broadcasted_iota(jnp.int32, (tb, num_mask, p), 2)
    oh_m = (mask_idx[:, :, None] == iota3).astype(jnp.float32)
    masked_pos = jnp.dot(oh_m.reshape(tb * num_mask, p), pos_ref[...],
                         preferred_element_type=jnp.float32, precision=_HIGH)
    mtproj = jnp.dot(mt_ref[...], w_ref[...],
                     preferred_element_type=jnp.float32, precision=_HIGH)
    me_ref[...] = (masked_pos + mtproj + b_ref[...]).reshape(tb, num_mask, d)

    iota3u = jax.lax.broadcasted_iota(jnp.int32, (tb, num_unmask, p), 2)
    oh_u = (unmask_idx[:, :, None] == iota3u).astype(jnp.float32)
    up = jnp.dot(oh_u.reshape(tb * num_unmask, p), pos_ref[...],
                 preferred_element_type=jnp.float32, precision=_HIGH)
    up_ref[...] = up.reshape(tb, num_unmask, d)


def kernel(patches, W, b, pos_table, mask_token):
    bc, p, a = patches.shape
    d = W.shape[1]
    num_mask = int(0.75 * p)
    num_unmask = p - num_mask

    rand = jax.random.uniform(jax.random.key(42), (bc, p))

    tb = 32
    grid = bc // tb

    out_shapes = (
        jax.ShapeDtypeStruct((bc, num_unmask, d), jnp.float32),
        jax.ShapeDtypeStruct((bc, num_mask, d), jnp.float32),
        jax.ShapeDtypeStruct((bc, num_unmask, d), jnp.float32),
        jax.ShapeDtypeStruct((bc, num_mask), jnp.int32),
        jax.ShapeDtypeStruct((bc, num_unmask), jnp.int32),
    )
    out_specs = (
        pl.BlockSpec((tb, num_unmask, d), lambda i: (i, 0, 0)),
        pl.BlockSpec((tb, num_mask, d), lambda i: (i, 0, 0)),
        pl.BlockSpec((tb, num_unmask, d), lambda i: (i, 0, 0)),
        pl.BlockSpec((tb, num_mask), lambda i: (i, 0)),
        pl.BlockSpec((tb, num_unmask), lambda i: (i, 0)),
    )
    in_specs = [
        pl.BlockSpec((tb, p), lambda i: (i, 0)),
        pl.BlockSpec((tb, p, a), lambda i: (i, 0, 0)),
        pl.BlockSpec((a, d), lambda i: (0, 0)),
        pl.BlockSpec((1, d), lambda i: (0, 0)),
        pl.BlockSpec((p, d), lambda i: (0, 0)),
        pl.BlockSpec((1, a), lambda i: (0, 0)),
    ]

    return pl.pallas_call(
        functools.partial(_tc_body, num_mask=num_mask),
        grid=(grid,),
        in_specs=in_specs,
        out_specs=out_specs,
        out_shape=out_shapes,
    )(rand, patches, W, b.reshape(1, d), pos_table, mask_token)


# speedup vs baseline: 5.4471x; 5.4471x over previous
import functools

import jax
import jax.numpy as jnp
from jax import lax
from jax.experimental import pallas as pl
from jax.experimental.pallas import tpu as pltpu
from jax.experimental.pallas import tpu_sc as plsc

_HIGH = jax.lax.Precision.HIGHEST


def _idx_body(rand_ref, w_ref, b_ref, pos_ref, mt_ref,
              me_ref, mi_ref, ui_ref, flat_ref, ranks_ref, *, num_mask):
    tb, p = rand_ref.shape
    d = pos_ref.shape[1]
    num_unmask = p - num_mask
    rand = rand_ref[...]

    rk = rand[:, :, None]
    rq = rand[:, None, :]
    ik = lax.broadcasted_iota(jnp.int32, (tb, p, p), 1)
    iq = lax.broadcasted_iota(jnp.int32, (tb, p, p), 2)
    before = ((rk < rq) | ((rk == rq) & (ik < iq))).astype(jnp.float32)
    ranks = jnp.sum(before, axis=1)
    ranks_ref[...] = ranks

    iota_m = lax.broadcasted_iota(jnp.int32, (tb, num_mask, p), 1).astype(jnp.float32)
    oh_m = (ranks[:, None, :] == iota_m).astype(jnp.float32)
    iota_u = lax.broadcasted_iota(jnp.int32, (tb, num_unmask, p), 1).astype(jnp.float32) + num_mask
    oh_u = (ranks[:, None, :] == iota_u).astype(jnp.float32)

    lane_q_m = lax.broadcasted_iota(jnp.int32, (tb, num_mask, p), 2).astype(jnp.float32)
    mask_idx = jnp.sum(oh_m * lane_q_m, axis=2).astype(jnp.int32)
    lane_q_u = lax.broadcasted_iota(jnp.int32, (tb, num_unmask, p), 2).astype(jnp.float32)
    unmask_idx = jnp.sum(oh_u * lane_q_u, axis=2).astype(jnp.int32)
    mi_ref[...] = mask_idx
    ui_ref[...] = unmask_idx

    base = pl.program_id(0) * tb
    row = lax.broadcasted_iota(jnp.int32, (tb, num_unmask), 0) + base
    flat_ref[...] = unmask_idx + p * row

    mtproj = jnp.dot(mt_ref[...], w_ref[...],
                     preferred_element_type=jnp.float32, precision=_HIGH)
    mpos = jnp.dot(oh_m.reshape(tb * num_mask, p), pos_ref[...],
                   preferred_element_type=jnp.float32, precision=_HIGH)
    me_ref[...] = (mpos + mtproj + b_ref[...]).reshape(tb, num_mask, d)


def _proj_body(cp_ref, ranks_ref, w_ref, b_ref, pos_ref,
               ue_ref, up_ref, *, num_mask):
    tb, p = ranks_ref.shape
    d = w_ref.shape[1]
    num_unmask = p - num_mask
    ranks = ranks_ref[...]

    iota_u = lax.broadcasted_iota(jnp.int32, (tb, num_unmask, p), 1).astype(jnp.float32) + num_mask
    oh_u = (ranks[:, None, :] == iota_u).astype(jnp.float32)
    upos = jnp.dot(oh_u.reshape(tb * num_unmask, p), pos_ref[...],
                   preferred_element_type=jnp.float32, precision=_HIGH)
    up_ref[...] = upos.reshape(tb, num_unmask, d)

    proj = jnp.dot(cp_ref[...], w_ref[...],
                   preferred_element_type=jnp.float32, precision=_HIGH)
    ue_ref[...] = (proj + upos + b_ref[...]).reshape(tb, num_unmask, d)


def _make_sc_gather(total_rows, a_dim, rows_per_w, chunk):
    mesh = plsc.VectorSubcoreMesh(core_axis_name="c", subcore_axis_name="s")

    @functools.partial(
        pl.kernel, mesh=mesh,
        out_type=jax.ShapeDtypeStruct((total_rows, a_dim), jnp.float32),
        compiler_params=pltpu.CompilerParams(use_tc_tiling_on_sc=False),
        scratch_types=[
            pltpu.VMEM((chunk,), jnp.int32),
            pltpu.VMEM((chunk, a_dim), jnp.float32),
            pltpu.SemaphoreType.DMA,
        ],
    )
    def gk(src_hbm, idx_hbm, out_hbm, idx_v, rows_v, sem):
        wid = lax.axis_index("s") * 2 + lax.axis_index("c")
        base = wid * rows_per_w
        for c in range(rows_per_w // chunk):
            off = base + c * chunk
            pltpu.sync_copy(idx_hbm.at[pl.ds(off, chunk)], idx_v)
            pltpu.async_copy(src_hbm.at[idx_v], rows_v, sem).wait()
            pltpu.sync_copy(rows_v, out_hbm.at[pl.ds(off, chunk)])

    return gk


def kernel(patches, W, b, pos_table, mask_token):
    bc, p, a = patches.shape
    d = W.shape[1]
    num_mask = int(0.75 * p)
    num_unmask = p - num_mask

    rand = jax.random.uniform(jax.random.key(42), (bc, p))
    b2 = b.reshape(1, d)

    tb1 = 64
    me, mi, ui, flat, ranks = pl.pallas_call(
        functools.partial(_idx_body, num_mask=num_mask),
        grid=(bc // tb1,),
        in_specs=[
            pl.BlockSpec((tb1, p), lambda i: (i, 0)),
            pl.BlockSpec((a, d), lambda i: (0, 0)),
            pl.BlockSpec((1, d), lambda i: (0, 0)),
            pl.BlockSpec((p, d), lambda i: (0, 0)),
            pl.BlockSpec((1, a), lambda i: (0, 0)),
        ],
        out_specs=(
            pl.BlockSpec((tb1, num_mask, d), lambda i: (i, 0, 0)),
            pl.BlockSpec((tb1, num_mask), lambda i: (i, 0)),
            pl.BlockSpec((tb1, num_unmask), lambda i: (i, 0)),
            pl.BlockSpec((tb1, num_unmask), lambda i: (i, 0)),
            pl.BlockSpec((tb1, p), lambda i: (i, 0)),
        ),
        out_shape=(
            jax.ShapeDtypeStruct((bc, num_mask, d), jnp.float32),
            jax.ShapeDtypeStruct((bc, num_mask), jnp.int32),
            jax.ShapeDtypeStruct((bc, num_unmask), jnp.int32),
            jax.ShapeDtypeStruct((bc, num_unmask), jnp.int32),
            jax.ShapeDtypeStruct((bc, p), jnp.float32),
        ),
    )(rand, W, b2, pos_table, mask_token)

    total = bc * num_unmask
    rows_per_w = total // 32
    chunk = min(rows_per_w, 1024)
    gk = _make_sc_gather(total, a, rows_per_w, chunk)
    cp = gk(patches.reshape(bc * p, a), flat.reshape(total))

    tb3 = 256
    ue, up = pl.pallas_call(
        functools.partial(_proj_body, num_mask=num_mask),
        grid=(bc // tb3,),
        in_specs=[
            pl.BlockSpec((tb3 * num_unmask, a), lambda i: (i, 0)),
            pl.BlockSpec((tb3, p), lambda i: (i, 0)),
            pl.BlockSpec((a, d), lambda i: (0, 0)),
            pl.BlockSpec((1, d), lambda i: (0, 0)),
            pl.BlockSpec((p, d), lambda i: (0, 0)),
        ],
        out_specs=(
            pl.BlockSpec((tb3, num_unmask, d), lambda i: (i, 0, 0)),
            pl.BlockSpec((tb3, num_unmask, d), lambda i: (i, 0, 0)),
        ),
        out_shape=(
            jax.ShapeDtypeStruct((bc, num_unmask, d), jnp.float32),
            jax.ShapeDtypeStruct((bc, num_unmask, d), jnp.float32),
        ),
    )(cp, ranks, W, b2, pos_table)

    return ue, me, up, mi, ui


# speedup vs baseline: 8.2779x; 1.5197x over previous
import functools

import jax
import jax.numpy as jnp
from jax import lax
from jax.experimental import pallas as pl
from jax.experimental.pallas import tpu as pltpu
from jax.experimental.pallas import tpu_sc as plsc

_HIGH = jax.lax.Precision.HIGHEST


def _main_body(rand_ref, patches_ref, w_ref, b_ref, pos_ref, mt_ref,
               pe_ref, me_ref, up_ref, mi_ref, ui_ref, flat_ref, *, num_mask):
    tb, p = rand_ref.shape
    a = patches_ref.shape[2]
    d = w_ref.shape[1]
    num_unmask = p - num_mask
    rand = rand_ref[...]

    rk = rand[:, :, None]
    rq = rand[:, None, :]
    ik = lax.broadcasted_iota(jnp.int32, (tb, p, p), 1)
    iq = lax.broadcasted_iota(jnp.int32, (tb, p, p), 2)
    before = ((rk < rq) | ((rk == rq) & (ik < iq))).astype(jnp.float32)
    ranks = jnp.sum(before, axis=1)

    iota_i = lax.broadcasted_iota(jnp.int32, (tb, p, p), 1).astype(jnp.float32)
    oh = (ranks[:, None, :] == iota_i).astype(jnp.float32)

    perm = jnp.dot(oh.reshape(tb * p, p), pos_ref[...],
                   preferred_element_type=jnp.float32,
                   precision=_HIGH).reshape(tb, p, d)

    mtproj = jnp.dot(mt_ref[...], w_ref[...],
                     preferred_element_type=jnp.float32, precision=_HIGH)
    me_ref[...] = perm[:, :num_mask, :] + (mtproj + b_ref[...])[None]
    up_ref[...] = perm[:, num_mask:, :]

    lane_q = lax.broadcasted_iota(jnp.int32, (tb, p, p), 2).astype(jnp.float32)
    inv = jnp.sum(oh * lane_q, axis=2).astype(jnp.int32)
    mi_ref[...] = inv[:, :num_mask]
    ui = inv[:, num_mask:]
    ui_ref[...] = ui

    base = pl.program_id(0) * tb
    row = lax.broadcasted_iota(jnp.int32, (tb, num_unmask), 0) + base
    flat_ref[...] = ui + p * row

    proj = jnp.dot(patches_ref[...].reshape(tb * p, a), w_ref[...],
                   preferred_element_type=jnp.float32)
    pe = proj.reshape(tb, p, d) + b_ref[...][None] + pos_ref[...][None]
    pe_ref[...] = pe.reshape(tb * p, d)


def _make_sc_gather(total_rows, d_dim, rows_per_w, chunk):
    mesh = plsc.VectorSubcoreMesh(core_axis_name="c", subcore_axis_name="s")

    @functools.partial(
        pl.kernel, mesh=mesh,
        out_type=jax.ShapeDtypeStruct((total_rows, d_dim), jnp.float32),
        scratch_types=[
            pltpu.VMEM((chunk,), jnp.int32),
            pltpu.VMEM((chunk, d_dim), jnp.float32),
            pltpu.SemaphoreType.DMA,
        ],
    )
    def gk(src_hbm, idx_hbm, out_hbm, idx_v, rows_v, sem):
        wid = lax.axis_index("s") * 2 + lax.axis_index("c")
        base = wid * rows_per_w
        for c in range(rows_per_w // chunk):
            off = base + c * chunk
            pltpu.sync_copy(idx_hbm.at[pl.ds(off, chunk)], idx_v)
            pltpu.async_copy(src_hbm.at[idx_v], rows_v, sem).wait()
            pltpu.sync_copy(rows_v, out_hbm.at[pl.ds(off, chunk)])

    return gk


def kernel(patches, W, b, pos_table, mask_token):
    bc, p, a = patches.shape
    d = W.shape[1]
    num_mask = int(0.75 * p)
    num_unmask = p - num_mask

    rand = jax.random.uniform(jax.random.key(42), (bc, p))
    b2 = b.reshape(1, d)

    tb = 64
    pe, me, up, mi, ui, flat = pl.pallas_call(
        functools.partial(_main_body, num_mask=num_mask),
        grid=(bc // tb,),
        in_specs=[
            pl.BlockSpec((tb, p), lambda i: (i, 0)),
            pl.BlockSpec((tb, p, a), lambda i: (i, 0, 0)),
            pl.BlockSpec((a, d), lambda i: (0, 0)),
            pl.BlockSpec((1, d), lambda i: (0, 0)),
            pl.BlockSpec((p, d), lambda i: (0, 0)),
            pl.BlockSpec((1, a), lambda i: (0, 0)),
        ],
        out_specs=(
            pl.BlockSpec((tb * p, d), lambda i: (i, 0)),
            pl.BlockSpec((tb, num_mask, d), lambda i: (i, 0, 0)),
            pl.BlockSpec((tb, num_unmask, d), lambda i: (i, 0, 0)),
            pl.BlockSpec((tb, num_mask), lambda i: (i, 0)),
            pl.BlockSpec((tb, num_unmask), lambda i: (i, 0)),
            pl.BlockSpec((tb, num_unmask), lambda i: (i, 0)),
        ),
        out_shape=(
            jax.ShapeDtypeStruct((bc * p, d), jnp.float32),
            jax.ShapeDtypeStruct((bc, num_mask, d), jnp.float32),
            jax.ShapeDtypeStruct((bc, num_unmask, d), jnp.float32),
            jax.ShapeDtypeStruct((bc, num_mask), jnp.int32),
            jax.ShapeDtypeStruct((bc, num_unmask), jnp.int32),
            jax.ShapeDtypeStruct((bc, num_unmask), jnp.int32),
        ),
    )(rand, patches, W, b2, pos_table, mask_token)

    total = bc * num_unmask
    rows_per_w = total // 32
    chunk = min(rows_per_w, 512)
    gk = _make_sc_gather(total, d, rows_per_w, chunk)
    ue = gk(pe, flat.reshape(total)).reshape(bc, num_unmask, d)

    return ue, me, up, mi, ui


# speedup vs baseline: 8.4132x; 1.0163x over previous
import functools

import jax
import jax.numpy as jnp
from jax import lax
from jax.experimental import pallas as pl
from jax.experimental.pallas import tpu as pltpu
from jax.experimental.pallas import tpu_sc as plsc

_HIGH = jax.lax.Precision.HIGHEST


def _main_body(rand_ref, patches_ref, w_ref, b_ref, pos_ref, mt_ref,
               pe_ref, me_ref, up_ref, mi_ref, ui_ref, flat_ref, *, num_mask):
    tb, p = rand_ref.shape
    a = patches_ref.shape[2]
    d = w_ref.shape[1]
    num_unmask = p - num_mask
    rand = rand_ref[...]

    rk = rand[:, :, None]
    rq = rand[:, None, :]
    ik = lax.broadcasted_iota(jnp.int32, (tb, p, p), 1)
    iq = lax.broadcasted_iota(jnp.int32, (tb, p, p), 2)
    before = ((rk < rq) | ((rk == rq) & (ik < iq))).astype(jnp.float32)
    ranks = jnp.sum(before, axis=1)

    iota_i = lax.broadcasted_iota(jnp.int32, (tb, p, p), 1).astype(jnp.float32)
    oh = (ranks[:, None, :] == iota_i).astype(jnp.float32)

    perm = jnp.dot(oh.reshape(tb * p, p), pos_ref[...],
                   preferred_element_type=jnp.float32,
                   precision=_HIGH).reshape(tb, p, d)

    mtproj = jnp.dot(mt_ref[...], w_ref[...],
                     preferred_element_type=jnp.float32, precision=_HIGH)
    me_ref[...] = perm[:, :num_mask, :] + (mtproj + b_ref[...])[None]
    up_ref[...] = perm[:, num_mask:, :]

    lane_q = lax.broadcasted_iota(jnp.int32, (tb, p, p), 2).astype(jnp.float32)
    inv = jnp.sum(oh * lane_q, axis=2).astype(jnp.int32)
    mi_ref[...] = inv[:, :num_mask]
    ui = inv[:, num_mask:]
    ui_ref[...] = ui

    base = pl.program_id(0) * tb
    row = lax.broadcasted_iota(jnp.int32, (tb, num_unmask), 0) + base
    flat_ref[...] = ui + p * row

    proj = jnp.dot(patches_ref[...].reshape(tb * p, a), w_ref[...],
                   preferred_element_type=jnp.float32)
    pe = proj.reshape(tb, p, d) + b_ref[...][None] + pos_ref[...][None]
    pe_ref[...] = pe.reshape(tb * p, d)


def _make_sc_gather(total_rows, d_dim, rows_per_w, chunk):
    mesh = plsc.VectorSubcoreMesh(core_axis_name="c", subcore_axis_name="s")

    @functools.partial(
        pl.kernel, mesh=mesh,
        out_type=jax.ShapeDtypeStruct((total_rows, d_dim), jnp.float32),
        scratch_types=[
            pltpu.VMEM((chunk,), jnp.int32),
            pltpu.VMEM((chunk, d_dim), jnp.float32),
            pltpu.SemaphoreType.DMA,
        ],
    )
    def gk(src_hbm, idx_hbm, out_hbm, idx_v, rows_v, sem):
        wid = lax.axis_index("s") * 2 + lax.axis_index("c")
        base = wid * rows_per_w
        for c in range(rows_per_w // chunk):
            off = base + c * chunk
            pltpu.sync_copy(idx_hbm.at[pl.ds(off, chunk)], idx_v)
            pltpu.async_copy(src_hbm.at[idx_v], rows_v, sem).wait()
            pltpu.sync_copy(rows_v, out_hbm.at[pl.ds(off, chunk)])

    return gk


def kernel(patches, W, b, pos_table, mask_token):
    bc, p, a = patches.shape
    d = W.shape[1]
    num_mask = int(0.75 * p)
    num_unmask = p - num_mask

    rand = jax.random.uniform(jax.random.key(42), (bc, p))
    b2 = b.reshape(1, d)

    tb = 128
    pe, me, up, mi, ui, flat = pl.pallas_call(
        functools.partial(_main_body, num_mask=num_mask),
        grid=(bc // tb,),
        in_specs=[
            pl.BlockSpec((tb, p), lambda i: (i, 0)),
            pl.BlockSpec((tb, p, a), lambda i: (i, 0, 0)),
            pl.BlockSpec((a, d), lambda i: (0, 0)),
            pl.BlockSpec((1, d), lambda i: (0, 0)),
            pl.BlockSpec((p, d), lambda i: (0, 0)),
            pl.BlockSpec((1, a), lambda i: (0, 0)),
        ],
        out_specs=(
            pl.BlockSpec((tb * p, d), lambda i: (i, 0)),
            pl.BlockSpec((tb, num_mask, d), lambda i: (i, 0, 0)),
            pl.BlockSpec((tb, num_unmask, d), lambda i: (i, 0, 0)),
            pl.BlockSpec((tb, num_mask), lambda i: (i, 0)),
            pl.BlockSpec((tb, num_unmask), lambda i: (i, 0)),
            pl.BlockSpec((tb, num_unmask), lambda i: (i, 0)),
        ),
        out_shape=(
            jax.ShapeDtypeStruct((bc * p, d), jnp.float32),
            jax.ShapeDtypeStruct((bc, num_mask, d), jnp.float32),
            jax.ShapeDtypeStruct((bc, num_unmask, d), jnp.float32),
            jax.ShapeDtypeStruct((bc, num_mask), jnp.int32),
            jax.ShapeDtypeStruct((bc, num_unmask), jnp.int32),
            jax.ShapeDtypeStruct((bc, num_unmask), jnp.int32),
        ),
    )(rand, patches, W, b2, pos_table, mask_token)

    total = bc * num_unmask
    rows_per_w = total // 32
    chunk = min(rows_per_w, 512)
    gk = _make_sc_gather(total, d, rows_per_w, chunk)
    ue = gk(pe, flat.reshape(total)).reshape(bc, num_unmask, d)

    return ue, me, up, mi, ui
